# baseline (speedup 1.0000x reference)
import functools

import jax
import jax.numpy as jnp
from jax import lax
from jax.experimental import pallas as pl
from jax.experimental.pallas import tpu as pltpu
from jax.experimental.pallas import tpu_sc as plsc

_NC, _NS = 2, 16
_NW = _NC * _NS

_N_ATOMS = 10000
_N_BONDS = 320000
_MAX_NB = 32
_H = 128
_BOND_FDIM = 144
_N_MOLS = 100
_APM = _N_ATOMS // _N_MOLS

_AT_PER_W = 320
_ATOMS_PAD = _AT_PER_W * _NW
_GS_NB = 8
_GS_NBATCH = _AT_PER_W // _GS_NB
_GS_IDXROWS_W = _AT_PER_W * _MAX_NB // 128

_GB_CHUNKS_W = 79
_GB_CHUNKS = _GB_CHUNKS_W * _NW
_BONDS_PAD = _GB_CHUNKS * 128

_MESH = plsc.VectorSubcoreMesh(core_axis_name="c", subcore_axis_name="s")


def _wid():
    return lax.axis_index("s") * _NC + lax.axis_index("c")


def _gs(table, a2b2d):
    @functools.partial(
        pl.kernel,
        out_type=jax.ShapeDtypeStruct((_ATOMS_PAD, _H), jnp.float32),
        mesh=_MESH,
        scratch_types=[
            pltpu.VMEM((2, 128), jnp.int32),
            pltpu.VMEM((2, 128), jnp.int32),
            pltpu.VMEM((_GS_NB * _MAX_NB, _H), jnp.float32),
            pltpu.VMEM((_GS_NB * _MAX_NB, _H), jnp.float32),
            pltpu.VMEM((_GS_NB, _H), jnp.float32),
            pltpu.SemaphoreType.DMA,
            pltpu.SemaphoreType.DMA,
        ],
    )
    def k(table_h, a2b_h, out_h, idx0, idx1, rows0, rows1, acc, sem0, sem1):
        w = _wid()
        idx_bufs = (idx0, idx1)
        row_bufs = (rows0, rows1)
        sems = (sem0, sem1)
        idx_base = w * _GS_IDXROWS_W
        atom_base = w * _AT_PER_W

        def fire(bi, b):
            r0 = idx_base + bi * 2
            pltpu.sync_copy(a2b_h.at[pl.ds(r0, 2)], idx_bufs[b])
            pltpu.async_copy(table_h.at[idx_bufs[b].at[0]],
                             row_bufs[b].at[pl.ds(0, 128)], sems[b])
            pltpu.async_copy(table_h.at[idx_bufs[b].at[1]],
                             row_bufs[b].at[pl.ds(128, 128)], sems[b])

        def drain(b):
            pltpu.make_async_copy(table_h.at[idx_bufs[b].at[0]],
                                  row_bufs[b].at[pl.ds(0, 128)], sems[b]).wait()
            pltpu.make_async_copy(table_h.at[idx_bufs[b].at[1]],
                                  row_bufs[b].at[pl.ds(128, 128)], sems[b]).wait()

        def reduce_store(bi, b):
            rows = row_bufs[b]

            def red(i, _):
                base = i * _MAX_NB
                for g in range(_H // 16):
                    go = g * 16
                    vals = [rows[base + kk, pl.ds(go, 16)]
                            for kk in range(_MAX_NB)]
                    while len(vals) > 1:
                        nxt = [vals[t] + vals[t + 1]
                               for t in range(0, len(vals) - 1, 2)]
                        if len(vals) % 2:
                            nxt.append(vals[-1])
                        vals = nxt
                    acc[i, pl.ds(go, 16)] = vals[0]
                return 0

            lax.fori_loop(0, _GS_NB, red, 0)
            a0 = atom_base + bi * _GS_NB
            pltpu.sync_copy(acc, out_h.at[pl.ds(a0, _GS_NB)])

        fire(0, 0)

        def body(t, _):
            for b in range(2):
                bi = 2 * t + b
                nxt = bi + 1

                @pl.when(nxt < _GS_NBATCH)
                def _():
                    fire(nxt, 1 - b)

                drain(b)
                reduce_store(bi, b)
            return 0

        lax.fori_loop(0, _GS_NBATCH // 2, body, 0)

    return k(table, a2b2d)


def _gb(amw, b2a2d):
    @functools.partial(
        pl.kernel,
        out_type=jax.ShapeDtypeStruct((_BONDS_PAD, _H), jnp.float32),
        mesh=_MESH,
        scratch_types=[
            pltpu.VMEM((1, 128), jnp.int32),
            pltpu.VMEM((1, 128), jnp.int32),
            pltpu.VMEM((128, _H), jnp.float32),
            pltpu.VMEM((128, _H), jnp.float32),
            pltpu.SemaphoreType.DMA,
            pltpu.SemaphoreType.DMA,
        ],
    )
    def k(amw_h, b2a_h, out_h, idx0, idx1, rows0, rows1, sem0, sem1):
        w = _wid()
        idx_bufs = (idx0, idx1)
        row_bufs = (rows0, rows1)
        sems = (sem0, sem1)
        base = w * _GB_CHUNKS_W

        def fire(c, b):
            r = base + c
            pltpu.sync_copy(b2a_h.at[pl.ds(r, 1)], idx_bufs[b])
            pltpu.async_copy(amw_h.at[idx_bufs[b].at[0]], row_bufs[b], sems[b])

        def drain(b):
            pltpu.make_async_copy(amw_h.at[idx_bufs[b].at[0]], row_bufs[b],
                                  sems[b]).wait()

        def store(c, b):
            r = base + c
            pltpu.sync_copy(row_bufs[b], out_h.at[pl.ds(r * 128, 128)])

        fire(0, 0)

        def body(t, _):
            for b in range(2):
                c = 2 * t + b
                fire(c + 1, 1 - b)
                drain(b)
                store(c, b)
            return 0

        lax.fori_loop(0, (_GB_CHUNKS_W - 1) // 2, body, 0)
        drain(0)
        store(_GB_CHUNKS_W - 1, 0)

    return k(amw, b2a2d)


_BR = 512


def _pairswap(x):
    up = jnp.concatenate([x[1:], x[:1]], axis=0)
    dn = jnp.concatenate([x[-1:], x[:-1]], axis=0)
    par = lax.broadcasted_iota(jnp.int32, x.shape, 0) % 2
    return jnp.where(par == 0, up, dn)


def _k0_body(fb_ref, wiT_ref, whT_ref, inp_ref, u0_ref):
    inp = jnp.dot(fb_ref[...], wiT_ref[...], preferred_element_type=jnp.float32)
    m = jnp.maximum(inp, 0.0)
    inp_ref[...] = inp
    u0_ref[...] = jnp.dot(m, whT_ref[...], preferred_element_type=jnp.float32)


def _k0(fb, wiT, whT):
    return pl.pallas_call(
        _k0_body,
        grid=(_N_BONDS // _BR,),
        in_specs=[
            pl.BlockSpec((_BR, _BOND_FDIM), lambda i: (i, 0)),
            pl.BlockSpec((_BOND_FDIM, _H), lambda i: (0, 0)),
            pl.BlockSpec((_H, _H), lambda i: (0, 0)),
        ],
        out_specs=[pl.BlockSpec((_BR, _H), lambda i: (i, 0))] * 2,
        out_shape=[jax.ShapeDtypeStruct((_N_BONDS, _H), jnp.float32)] * 2,
    )(fb, wiT, whT)


def _k1_body(inp_ref, g_ref, u_ref, whT_ref, out_ref):
    m = jnp.maximum(inp_ref[...] + g_ref[...] - _pairswap(u_ref[...]), 0.0)
    out_ref[...] = jnp.dot(m, whT_ref[...], preferred_element_type=jnp.float32)


def _k1(inp, g, u, whT):
    return pl.pallas_call(
        _k1_body,
        grid=(_N_BONDS // _BR,),
        in_specs=[
            pl.BlockSpec((_BR, _H), lambda i: (i, 0)),
            pl.BlockSpec((_BR, _H), lambda i: (i, 0)),
            pl.BlockSpec((_BR, _H), lambda i: (i, 0)),
            pl.BlockSpec((_H, _H), lambda i: (0, 0)),
        ],
        out_specs=pl.BlockSpec((_BR, _H), lambda i: (i, 0)),
        out_shape=jax.ShapeDtypeStruct((_N_BONDS, _H), jnp.float32),
    )(inp, g, u, whT)


def _k2_body(inp_ref, g_ref, u_ref, out_ref):
    out_ref[...] = jnp.maximum(
        inp_ref[...] + g_ref[...] - _pairswap(u_ref[...]), 0.0)


def _k2(inp, g, u):
    return pl.pallas_call(
        _k2_body,
        grid=(_N_BONDS // _BR,),
        in_specs=[
            pl.BlockSpec((_BR, _H), lambda i: (i, 0)),
            pl.BlockSpec((_BR, _H), lambda i: (i, 0)),
            pl.BlockSpec((_BR, _H), lambda i: (i, 0)),
        ],
        out_specs=pl.BlockSpec((_BR, _H), lambda i: (i, 0)),
        out_shape=jax.ShapeDtypeStruct((_N_BONDS, _H), jnp.float32),
    )(inp, g, u)


def _k3_body(fa_ref, a3_ref, w1_ref, w2_ref, bo_ref, out_ref):
    h = jnp.maximum(
        jnp.dot(fa_ref[...], w1_ref[...], preferred_element_type=jnp.float32)
        + jnp.dot(a3_ref[...], w2_ref[...], preferred_element_type=jnp.float32)
        + bo_ref[...], 0.0)
    mol = lax.broadcasted_iota(jnp.int32, (_N_MOLS, _N_ATOMS), 0)
    row = lax.broadcasted_iota(jnp.int32, (_N_MOLS, _N_ATOMS), 1) // _APM
    sel = jnp.where(mol == row, 1.0 / _APM, 0.0)
    out_ref[...] = jnp.dot(sel, h, preferred_element_type=jnp.float32)


def _k3(fa, a3, w1T, w2T, bo):
    return pl.pallas_call(
        _k3_body,
        in_specs=[
            pl.BlockSpec((_N_ATOMS, _H), lambda: (0, 0)),
            pl.BlockSpec((_N_ATOMS, _H), lambda: (0, 0)),
            pl.BlockSpec((_H, _H), lambda: (0, 0)),
            pl.BlockSpec((_H, _H), lambda: (0, 0)),
            pl.BlockSpec((1, _H), lambda: (0, 0)),
        ],
        out_specs=pl.BlockSpec((_N_MOLS, _H), lambda: (0, 0)),
        out_shape=jax.ShapeDtypeStruct((_N_MOLS, _H), jnp.float32),
    )(fa, a3, w1T, w2T, bo)


def kernel(f_atoms, f_bonds, a2b, b2a, b2revb, a_scope, W_i, W_h, W_o, b_o):
    del b2revb, a_scope
    wiT = W_i.T
    whT = W_h.T
    w1T = W_o[:, :_H].T
    w2T = W_o[:, _H:].T
    bo = b_o.reshape(1, _H)

    a2b2d = jnp.pad(a2b, ((0, _ATOMS_PAD - _N_ATOMS), (0, 0))).reshape(
        _ATOMS_PAD * _MAX_NB // 128, 128)
    b2a2d = jnp.pad(b2a, (0, _BONDS_PAD - _N_BONDS)).reshape(_GB_CHUNKS, 128)

    inp, u0 = _k0(f_bonds, wiT, whT)
    amw0 = _gs(u0, a2b2d)
    g0 = _gb(amw0, b2a2d)
    u1 = _k1(inp, g0, u0, whT)
    amw1 = _gs(u1, a2b2d)
    g1 = _gb(amw1, b2a2d)
    m2 = _k2(inp, g1, u1)
    a3 = _gs(m2, a2b2d)
    return _k3(f_atoms, a3[:_N_ATOMS], w1T, w2T, bo)


# speedup vs baseline: 1.0128x; 1.0128x over previous
import functools

import jax
import jax.numpy as jnp
from jax import lax
from jax.experimental import pallas as pl
from jax.experimental.pallas import tpu as pltpu
from jax.experimental.pallas import tpu_sc as plsc

_NC, _NS = 2, 16
_NW = _NC * _NS

_N_ATOMS = 10000
_N_BONDS = 320000
_MAX_NB = 32
_H = 128
_BOND_FDIM = 144
_N_MOLS = 100
_APM = _N_ATOMS // _N_MOLS

_AT_PER_W = 320
_ATOMS_PAD = _AT_PER_W * _NW
_GS_NB = 8
_GS_NBATCH = _AT_PER_W // _GS_NB
_GS_IDXROWS_W = _AT_PER_W * _MAX_NB // 128

_GB_CHUNKS_W = 79
_GB_CHUNKS = _GB_CHUNKS_W * _NW
_BONDS_PAD = _GB_CHUNKS * 128

_MESH = plsc.VectorSubcoreMesh(core_axis_name="c", subcore_axis_name="s")


def _wid():
    return lax.axis_index("s") * _NC + lax.axis_index("c")


def _gs(table, a2b2d, dest3d, z320):
    @functools.partial(
        pl.kernel,
        out_type=jax.ShapeDtypeStruct((_ATOMS_PAD, _H), jnp.float32),
        mesh=_MESH,
        scratch_types=[
            pltpu.VMEM((2, 128), jnp.int32),
            pltpu.VMEM((2, 128), jnp.int32),
            pltpu.VMEM((_GS_NB * _MAX_NB, _H), jnp.float32),
            pltpu.VMEM((_GS_NB * _MAX_NB, _H), jnp.float32),
            pltpu.VMEM((2 * _GS_NBATCH, 128), jnp.int32),
            pltpu.VMEM_SHARED((_NS * _AT_PER_W, _H), jnp.float32),
            pltpu.SemaphoreType.DMA,
            pltpu.SemaphoreType.DMA,
        ],
    )
    def k(table_h, a2b_h, dest_h, z_h, out_h,
          idx0, idx1, rows0, rows1, dest_v, acc_sh, sem0, sem1):
        c = lax.axis_index("c")
        s = lax.axis_index("s")
        w = s * _NC + c
        idx_bufs = (idx0, idx1)
        row_bufs = (rows0, rows1)
        sems = (sem0, sem1)
        idx_base = w * _GS_IDXROWS_W

        pltpu.sync_copy(dest_h.at[s], dest_v)
        pltpu.sync_copy(z_h, acc_sh.at[pl.ds(s * _AT_PER_W, _AT_PER_W)])

        def fire(bi, b):
            r0 = idx_base + bi * 2
            pltpu.sync_copy(a2b_h.at[pl.ds(r0, 2)], idx_bufs[b])
            pltpu.async_copy(table_h.at[idx_bufs[b].at[0]],
                             row_bufs[b].at[pl.ds(0, 128)], sems[b])
            pltpu.async_copy(table_h.at[idx_bufs[b].at[1]],
                             row_bufs[b].at[pl.ds(128, 128)], sems[b])

        def drain(b):
            pltpu.make_async_copy(table_h.at[idx_bufs[b].at[0]],
                                  row_bufs[b].at[pl.ds(0, 128)], sems[b]).wait()
            pltpu.make_async_copy(table_h.at[idx_bufs[b].at[1]],
                                  row_bufs[b].at[pl.ds(128, 128)], sems[b]).wait()

        def scatter_add(bi, b):
            r = bi * 2
            pltpu.sync_copy(row_bufs[b].at[pl.ds(0, 128)],
                            acc_sh.at[dest_v.at[r]], add=True)
            pltpu.sync_copy(row_bufs[b].at[pl.ds(128, 128)],
                            acc_sh.at[dest_v.at[r + 1]], add=True)

        fire(0, 0)

        def body(t, _):
            for b in range(2):
                bi = 2 * t + b
                nxt = bi + 1

                @pl.when(nxt < _GS_NBATCH)
                def _():
                    fire(nxt, 1 - b)

                drain(b)
                scatter_add(bi, b)
            return 0

        lax.fori_loop(0, _GS_NBATCH // 2, body, 0)
        pltpu.sync_copy(acc_sh.at[pl.ds(s * _AT_PER_W, _AT_PER_W)],
                        out_h.at[pl.ds(w * _AT_PER_W, _AT_PER_W)])

    return k(table, a2b2d, dest3d, z320)


def _gb(amw, b2a2d):
    @functools.partial(
        pl.kernel,
        out_type=jax.ShapeDtypeStruct((_BONDS_PAD, _H), jnp.float32),
        mesh=_MESH,
        scratch_types=[
            pltpu.VMEM((1, 128), jnp.int32),
            pltpu.VMEM((1, 128), jnp.int32),
            pltpu.VMEM((128, _H), jnp.float32),
            pltpu.VMEM((128, _H), jnp.float32),
            pltpu.SemaphoreType.DMA,
            pltpu.SemaphoreType.DMA,
        ],
    )
    def k(amw_h, b2a_h, out_h, idx0, idx1, rows0, rows1, sem0, sem1):
        w = _wid()
        idx_bufs = (idx0, idx1)
        row_bufs = (rows0, rows1)
        sems = (sem0, sem1)
        base = w * _GB_CHUNKS_W

        def fire(c, b):
            r = base + c
            pltpu.sync_copy(b2a_h.at[pl.ds(r, 1)], idx_bufs[b])
            pltpu.async_copy(amw_h.at[idx_bufs[b].at[0]], row_bufs[b], sems[b])

        def drain(b):
            pltpu.make_async_copy(amw_h.at[idx_bufs[b].at[0]], row_bufs[b],
                                  sems[b]).wait()

        def store(c, b):
            r = base + c
            pltpu.sync_copy(row_bufs[b], out_h.at[pl.ds(r * 128, 128)])

        fire(0, 0)

        def body(t, _):
            for b in range(2):
                c = 2 * t + b
                fire(c + 1, 1 - b)
                drain(b)
                store(c, b)
            return 0

        lax.fori_loop(0, (_GB_CHUNKS_W - 1) // 2, body, 0)
        drain(0)
        store(_GB_CHUNKS_W - 1, 0)

    return k(amw, b2a2d)


_BR = 512


def _pairswap(x):
    up = jnp.concatenate([x[1:], x[:1]], axis=0)
    dn = jnp.concatenate([x[-1:], x[:-1]], axis=0)
    par = lax.broadcasted_iota(jnp.int32, x.shape, 0) % 2
    return jnp.where(par == 0, up, dn)


def _k0_body(fb_ref, wiT_ref, whT_ref, inp_ref, u0_ref):
    inp = jnp.dot(fb_ref[...], wiT_ref[...], preferred_element_type=jnp.float32)
    m = jnp.maximum(inp, 0.0)
    inp_ref[...] = inp
    u0_ref[...] = jnp.dot(m, whT_ref[...], preferred_element_type=jnp.float32)


def _k0(fb, wiT, whT):
    return pl.pallas_call(
        _k0_body,
        grid=(_N_BONDS // _BR,),
        in_specs=[
            pl.BlockSpec((_BR, _BOND_FDIM), lambda i: (i, 0)),
            pl.BlockSpec((_BOND_FDIM, _H), lambda i: (0, 0)),
            pl.BlockSpec((_H, _H), lambda i: (0, 0)),
        ],
        out_specs=[pl.BlockSpec((_BR, _H), lambda i: (i, 0))] * 2,
        out_shape=[jax.ShapeDtypeStruct((_N_BONDS, _H), jnp.float32)] * 2,
    )(fb, wiT, whT)


def _k1_body(inp_ref, g_ref, u_ref, whT_ref, out_ref):
    m = jnp.maximum(inp_ref[...] + g_ref[...] - _pairswap(u_ref[...]), 0.0)
    out_ref[...] = jnp.dot(m, whT_ref[...], preferred_element_type=jnp.float32)


def _k1(inp, g, u, whT):
    return pl.pallas_call(
        _k1_body,
        grid=(_N_BONDS // _BR,),
        in_specs=[
            pl.BlockSpec((_BR, _H), lambda i: (i, 0)),
            pl.BlockSpec((_BR, _H), lambda i: (i, 0)),
            pl.BlockSpec((_BR, _H), lambda i: (i, 0)),
            pl.BlockSpec((_H, _H), lambda i: (0, 0)),
        ],
        out_specs=pl.BlockSpec((_BR, _H), lambda i: (i, 0)),
        out_shape=jax.ShapeDtypeStruct((_N_BONDS, _H), jnp.float32),
    )(inp, g, u, whT)


def _k2_body(inp_ref, g_ref, u_ref, out_ref):
    out_ref[...] = jnp.maximum(
        inp_ref[...] + g_ref[...] - _pairswap(u_ref[...]), 0.0)


def _k2(inp, g, u):
    return pl.pallas_call(
        _k2_body,
        grid=(_N_BONDS // _BR,),
        in_specs=[
            pl.BlockSpec((_BR, _H), lambda i: (i, 0)),
            pl.BlockSpec((_BR, _H), lambda i: (i, 0)),
            pl.BlockSpec((_BR, _H), lambda i: (i, 0)),
        ],
        out_specs=pl.BlockSpec((_BR, _H), lambda i: (i, 0)),
        out_shape=jax.ShapeDtypeStruct((_N_BONDS, _H), jnp.float32),
    )(inp, g, u)


def _k3_body(fa_ref, a3_ref, w1_ref, w2_ref, bo_ref, out_ref):
    h = jnp.maximum(
        jnp.dot(fa_ref[...], w1_ref[...], preferred_element_type=jnp.float32)
        + jnp.dot(a3_ref[...], w2_ref[...], preferred_element_type=jnp.float32)
        + bo_ref[...], 0.0)
    mol = lax.broadcasted_iota(jnp.int32, (_N_MOLS, _N_ATOMS), 0)
    row = lax.broadcasted_iota(jnp.int32, (_N_MOLS, _N_ATOMS), 1) // _APM
    sel = jnp.where(mol == row, 1.0 / _APM, 0.0)
    out_ref[...] = jnp.dot(sel, h, preferred_element_type=jnp.float32)


def _k3(fa, a3, w1T, w2T, bo):
    return pl.pallas_call(
        _k3_body,
        in_specs=[
            pl.BlockSpec((_N_ATOMS, _H), lambda: (0, 0)),
            pl.BlockSpec((_N_ATOMS, _H), lambda: (0, 0)),
            pl.BlockSpec((_H, _H), lambda: (0, 0)),
            pl.BlockSpec((_H, _H), lambda: (0, 0)),
            pl.BlockSpec((1, _H), lambda: (0, 0)),
        ],
        out_specs=pl.BlockSpec((_N_MOLS, _H), lambda: (0, 0)),
        out_shape=jax.ShapeDtypeStruct((_N_MOLS, _H), jnp.float32),
    )(fa, a3, w1T, w2T, bo)


def kernel(f_atoms, f_bonds, a2b, b2a, b2revb, a_scope, W_i, W_h, W_o, b_o):
    del b2revb, a_scope
    wiT = W_i.T
    whT = W_h.T
    w1T = W_o[:, :_H].T
    w2T = W_o[:, _H:].T
    bo = b_o.reshape(1, _H)

    a2b2d = jnp.pad(a2b, ((0, _ATOMS_PAD - _N_ATOMS), (0, 0))).reshape(
        _ATOMS_PAD * _MAX_NB // 128, 128)
    b2a2d = jnp.pad(b2a, (0, _BONDS_PAD - _N_BONDS)).reshape(_GB_CHUNKS, 128)

    j = jnp.arange(_AT_PER_W * _MAX_NB, dtype=jnp.int32) // _MAX_NB
    dest3d = (jnp.arange(_NS, dtype=jnp.int32)[:, None] * _AT_PER_W
              + j[None, :]).reshape(_NS, 2 * _GS_NBATCH, 128)
    z320 = jnp.zeros((_AT_PER_W, _H), jnp.float32)

    inp, u0 = _k0(f_bonds, wiT, whT)
    amw0 = _gs(u0, a2b2d, dest3d, z320)
    g0 = _gb(amw0, b2a2d)
    u1 = _k1(inp, g0, u0, whT)
    amw1 = _gs(u1, a2b2d, dest3d, z320)
    g1 = _gb(amw1, b2a2d)
    m2 = _k2(inp, g1, u1)
    a3 = _gs(m2, a2b2d, dest3d, z320)
    return _k3(f_atoms, a3[:_N_ATOMS], w1T, w2T, bo)
